# Initial kernel scaffold; baseline (speedup 1.0000x reference)
import jax
import jax.numpy as jnp
from jax.experimental import pallas as pl


def kernel(h, x, permutation_weights, expert_weights, bias):
    raise NotImplementedError("write your pallas kernel here")


# speedup vs baseline: 2.3160x; 2.3160x over previous
import functools

import numpy as np
import jax
import jax.numpy as jnp
from jax import lax
from jax.experimental import pallas as pl
from jax.experimental.pallas import tpu as pltpu
from jax.experimental.pallas import tpu_sc as plsc

_E = 8
_L = 16

_SCHED = 1.0 - np.exp(-1.0 / 1.0)
_K = max(int(_E - np.floor(_SCHED * _E)), 1)


def _gate_body(ew_hbm, bias_hbm, permt_hbm, out_hbm, ew_v, bias_v, permt_v, out_v):
    wid = lax.axis_index("s") * 2 + lax.axis_index("c")

    @pl.when(wid == 0)
    def _():
        pltpu.sync_copy(ew_hbm, ew_v)
        pltpu.sync_copy(bias_hbm, bias_v)
        pltpu.sync_copy(permt_hbm, permt_v)

        lvv = ew_v[...] + bias_v[...]
        lv = [lvv[j] for j in range(_E)]

        one, zero = jnp.int32(1), jnp.int32(0)
        sel = []
        for j in range(_E):
            rank = zero
            for i in range(_E):
                if i == j:
                    continue
                ahead = lv[i] > lv[j]
                if i < j:
                    ahead = ahead | (lv[i] == lv[j])
                rank = rank + jnp.where(ahead, one, zero)
            sel.append(rank < _K)

        xs = [jnp.where(sel[j], lv[j], jnp.float32(-1e9)) for j in range(_E)]
        m = xs[0]
        for j in range(1, _E):
            m = jnp.maximum(m, xs[j])

        lanes = lax.iota(jnp.int32, _L)
        xv = jnp.full((_L,), -1e30, jnp.float32)
        for j in range(_E):
            xv = jnp.where(lanes == j, xs[j], xv)
        ev = jnp.exp(xv - m)

        s = ev[0]
        for j in range(1, _E):
            s = s + ev[j]

        acc = ev[0] * permt_v[0, :]
        for j in range(1, _E):
            acc = acc + ev[j] * permt_v[j, :]

        out_v[...] = acc / s
        pltpu.sync_copy(out_v, out_hbm)


@jax.jit
def _gate(ew16, bias16, permt):
    mesh = plsc.VectorSubcoreMesh(core_axis_name="c", subcore_axis_name="s")
    return pl.kernel(
        _gate_body,
        out_type=jax.ShapeDtypeStruct((_L,), jnp.float32),
        mesh=mesh,
        scratch_types=[
            pltpu.VMEM((_L,), jnp.float32),
            pltpu.VMEM((_L,), jnp.float32),
            pltpu.VMEM((_E, _L), jnp.float32),
            pltpu.VMEM((_L,), jnp.float32),
        ],
    )(ew16, bias16, permt)


def _combine_body(probs_ref, h_ref, o_ref):
    acc = probs_ref[0] * h_ref[0]
    for e in range(1, _E):
        acc = acc + probs_ref[e] * h_ref[e]
    o_ref[...] = acc


@jax.jit
def _combine(probs, h):
    E, T, D = h.shape
    tt = 512
    return pl.pallas_call(
        _combine_body,
        grid=(T // tt,),
        in_specs=[
            pl.BlockSpec(memory_space=pltpu.SMEM),
            pl.BlockSpec((E, tt, D), lambda i: (0, i, 0)),
        ],
        out_specs=pl.BlockSpec((tt, D), lambda i: (i, 0)),
        out_shape=jax.ShapeDtypeStruct((T, D), jnp.float32),
        compiler_params=pltpu.CompilerParams(
            dimension_semantics=("arbitrary",),
        ),
    )(probs, h)


def kernel(h, x, permutation_weights, expert_weights, bias):
    del x
    ew16 = jnp.pad(expert_weights[:, 0], (0, _L - _E))
    bias16 = jnp.pad(bias, (0, _L - _E))
    permt = jnp.pad(permutation_weights.T, ((0, 0), (0, _L - _E)))
    probs = _gate(ew16, bias16, permt)[:_E]
    return _combine(probs, h)


# speedup vs baseline: 2.3405x; 1.0106x over previous
import functools

import numpy as np
import jax
import jax.numpy as jnp
from jax import lax
from jax.experimental import pallas as pl
from jax.experimental.pallas import tpu as pltpu
from jax.experimental.pallas import tpu_sc as plsc

_E = 8
_L = 16

_SCHED = 1.0 - np.exp(-1.0 / 1.0)
_K = max(int(_E - np.floor(_SCHED * _E)), 1)


def _gate_body(ew_hbm, bias_hbm, permt_hbm, out_hbm, ew_v, bias_v, permt_v, out_v):
    wid = lax.axis_index("s") * 2 + lax.axis_index("c")

    @pl.when(wid == 0)
    def _():
        pltpu.sync_copy(ew_hbm, ew_v)
        pltpu.sync_copy(bias_hbm, bias_v)
        pltpu.sync_copy(permt_hbm, permt_v)

        lvv = ew_v[...] + bias_v[...]
        lv = [lvv[j] for j in range(_E)]

        one, zero = jnp.int32(1), jnp.int32(0)
        sel = []
        for j in range(_E):
            rank = zero
            for i in range(_E):
                if i == j:
                    continue
                ahead = lv[i] > lv[j]
                if i < j:
                    ahead = ahead | (lv[i] == lv[j])
                rank = rank + jnp.where(ahead, one, zero)
            sel.append(rank < _K)

        xs = [jnp.where(sel[j], lv[j], jnp.float32(-1e9)) for j in range(_E)]
        m = xs[0]
        for j in range(1, _E):
            m = jnp.maximum(m, xs[j])

        lanes = lax.iota(jnp.int32, _L)
        xv = jnp.full((_L,), -1e30, jnp.float32)
        for j in range(_E):
            xv = jnp.where(lanes == j, xs[j], xv)
        ev = jnp.exp(xv - m)

        s = ev[0]
        for j in range(1, _E):
            s = s + ev[j]

        acc = ev[0] * permt_v[0, :]
        for j in range(1, _E):
            acc = acc + ev[j] * permt_v[j, :]

        out_v[...] = acc / s
        pltpu.sync_copy(out_v, out_hbm)


@jax.jit
def _gate(ew16, bias16, permt):
    mesh = plsc.VectorSubcoreMesh(core_axis_name="c", subcore_axis_name="s")
    return pl.kernel(
        _gate_body,
        out_type=jax.ShapeDtypeStruct((_L,), jnp.float32),
        mesh=mesh,
        scratch_types=[
            pltpu.VMEM((_L,), jnp.float32),
            pltpu.VMEM((_L,), jnp.float32),
            pltpu.VMEM((_E, _L), jnp.float32),
            pltpu.VMEM((_L,), jnp.float32),
        ],
    )(ew16, bias16, permt)


def _combine_body(probs_ref, h_ref, o_ref):
    acc = probs_ref[0] * h_ref[0]
    for e in range(1, _E):
        acc = acc + probs_ref[e] * h_ref[e]
    o_ref[...] = acc


@jax.jit
def _combine(probs, h):
    E, T, D = h.shape
    tt = 256
    return pl.pallas_call(
        _combine_body,
        grid=(T // tt,),
        in_specs=[
            pl.BlockSpec(memory_space=pltpu.SMEM),
            pl.BlockSpec((E, tt, D), lambda i: (0, i, 0)),
        ],
        out_specs=pl.BlockSpec((tt, D), lambda i: (i, 0)),
        out_shape=jax.ShapeDtypeStruct((T, D), jnp.float32),
        compiler_params=pltpu.CompilerParams(
            dimension_semantics=("arbitrary",),
        ),
    )(probs, h)


def kernel(h, x, permutation_weights, expert_weights, bias):
    del x
    ew16 = jnp.pad(expert_weights[:, 0], (0, _L - _E))
    bias16 = jnp.pad(bias, (0, _L - _E))
    permt = jnp.pad(permutation_weights.T, ((0, 0), (0, _L - _E)))
    probs = _gate(ew16, bias16, permt)[:_E]
    return _combine(probs, h)
